# baseline (speedup 1.0000x reference)
import functools

import jax
import jax.numpy as jnp
from jax import lax
from jax.experimental import pallas as pl
from jax.experimental.pallas import tpu as pltpu
from jax.experimental.pallas import tpu_sc as plsc

D = 128
HID = 64
NCORE = 2
NSUB = 16
NW = NCORE * NSUB
CHUNK = 128
CW = 16
CWSC = 128


def _round_up(x, m):
    return (x + m - 1) // m * m


DEPTH = 4


def _seg_phase(table, src2, dst2, zeros_h, out, acc, wid, cid, sid,
               sidx, didx, rows0, rows1, g0, g1, t0, t1,
               n_dst_pad, e_pad):
    n_chunks = e_pad // NW // CHUNK
    sb = 40 if n_chunks % 40 == 0 else n_chunks
    n_sb = n_chunks // sb
    rps = n_dst_pad // NSUB

    pltpu.sync_copy(zeros_h, acc.at[pl.ds(sid * rps, rps)])
    plsc.subcore_barrier()

    @pl.loop(0, n_sb)
    def _(b):
        c0 = wid * n_chunks + b * sb
        pltpu.sync_copy(src2.at[pl.ds(c0, sb)], sidx.at[pl.ds(0, sb)])
        pltpu.sync_copy(dst2.at[pl.ds(c0, sb)], didx.at[pl.ds(0, sb)])

        pltpu.async_copy(table.at[sidx.at[0]], rows0, g0)
        pltpu.async_copy(table.at[sidx.at[1]], rows1, g1)

        @pl.loop(0, sb - 2, step=2)
        def _(j):
            pltpu.make_async_copy(table.at[sidx.at[j]], rows0, g0).wait()
            pltpu.async_copy(rows0, acc.at[didx.at[j]], t0, add=True)
            pltpu.make_async_copy(table.at[sidx.at[j + 1]], rows1, g1).wait()
            pltpu.async_copy(rows1, acc.at[didx.at[j + 1]], t1, add=True)
            pltpu.make_async_copy(rows0, acc.at[didx.at[j]], t0).wait()
            pltpu.async_copy(table.at[sidx.at[j + 2]], rows0, g0)
            pltpu.make_async_copy(rows1, acc.at[didx.at[j + 1]], t1).wait()
            pltpu.async_copy(table.at[sidx.at[j + 3]], rows1, g1)

        pltpu.make_async_copy(table.at[sidx.at[sb - 2]], rows0, g0).wait()
        pltpu.sync_copy(rows0, acc.at[didx.at[sb - 2]], add=True)
        pltpu.make_async_copy(table.at[sidx.at[sb - 1]], rows1, g1).wait()
        pltpu.sync_copy(rows1, acc.at[didx.at[sb - 1]], add=True)

    plsc.subcore_barrier()
    pltpu.sync_copy(acc.at[pl.ds(sid * rps, rps)],
                    out.at[pl.ds(cid * n_dst_pad + sid * rps, rps)])


_PH_V = (
    ("vals_to_cons", "vals", "cons"),
    ("vals_to_obj", "vals", "obj"),
)
_PH_O = (
    ("obj_to_vals", "obj", "vals"),
    ("obj_to_cons", "obj", "cons"),
)
_PH_C = (
    ("cons_to_vals", "cons", "vals"),
    ("cons_to_obj", "cons", "obj"),
)
_PHASES = _PH_V + _PH_O + _PH_C


@functools.lru_cache(maxsize=None)
def _make_round(np_map, e_pad_map, phases):
    np_map = dict(np_map)
    e_pad_map = dict(e_pad_map)
    mesh = plsc.VectorSubcoreMesh(core_axis_name="c", subcore_axis_name="s")
    big_rps = np_map["vals"] // NSUB
    small_rps = np_map["obj"] // NSUB
    n_ph = len(phases)
    has_small = any(np_map[dt] <= 128 for _, _, dt in phases)

    def body(*refs):
        t_cons, t_vals, t_obj = refs[0:3]
        idx = [(refs[3 + 2 * k], refs[4 + 2 * k]) for k in range(n_ph)]
        zeros_big, zeros_small = refs[3 + 2 * n_ph], refs[4 + 2 * n_ph]
        outs = refs[5 + 2 * n_ph:5 + 3 * n_ph]
        (sidx, didx, rows0, rows1, acc_big, acc_small,
         g0, g1, t0, t1) = refs[5 + 3 * n_ph:]
        cid = lax.axis_index("c")
        sid = lax.axis_index("s")
        wid = sid * NCORE + cid
        tables = {"cons": t_cons, "vals": t_vals, "obj": t_obj}
        for k, (name, st, dt) in enumerate(phases):
            big = np_map[dt] > 128
            _seg_phase(tables[st], idx[k][0], idx[k][1],
                       zeros_big if big else zeros_small, outs[k],
                       acc_big if big else acc_small, wid, cid, sid,
                       sidx, didx, rows0, rows1, g0, g1, t0, t1,
                       np_map[dt], e_pad_map[name])

    kern = pl.kernel(
        body,
        out_type=tuple(
            jax.ShapeDtypeStruct((NCORE * np_map[dt], D), jnp.float32)
            for _, _, dt in phases),
        mesh=mesh,
        scratch_types=[
            pltpu.VMEM((40, CHUNK), jnp.int32),
            pltpu.VMEM((40, CHUNK), jnp.int32),
            pltpu.VMEM((CHUNK, D), jnp.float32),
            pltpu.VMEM((CHUNK, D), jnp.float32),
            pltpu.VMEM_SHARED((np_map["vals"], D), jnp.float32),
            pltpu.VMEM_SHARED((np_map["obj"] if has_small else 8, D),
                              jnp.float32),
            pltpu.SemaphoreType.DMA,
            pltpu.SemaphoreType.DMA,
            pltpu.SemaphoreType.DMA,
            pltpu.SemaphoreType.DMA,
        ],
    )

    def call(h, prep):
        args = [h["cons"], h["vals"], h["obj"]]
        for name, st, dt in phases:
            src_p, dst_p, e_pad = prep[name]
            args.append(src_p.reshape(e_pad // CHUNK, CHUNK))
            args.append(dst_p.reshape(e_pad // CHUNK, CHUNK))
        args.append(jnp.zeros((big_rps, D), jnp.float32))
        args.append(jnp.zeros((small_rps, D), jnp.float32))
        outs = kern(*args)
        return {name: outs[k].reshape(NCORE, np_map[dt], D)
                for k, (name, st, dt) in enumerate(phases)}

    return call


def _cnt_phase(dst2, zeros_h, out, acc, ones_v, wid, cid, sid, didx,
               t0, t1, n_dst_pad, e_pad):
    n_chunks = e_pad // NW // CHUNK
    sb = 40 if n_chunks % 40 == 0 else n_chunks
    n_sb = n_chunks // sb
    rps = n_dst_pad // NSUB

    pltpu.sync_copy(zeros_h, acc.at[pl.ds(sid * rps, rps)])
    plsc.subcore_barrier()

    @pl.loop(0, n_sb)
    def _(b):
        c0 = wid * n_chunks + b * sb
        pltpu.sync_copy(dst2.at[pl.ds(c0, sb)], didx.at[pl.ds(0, sb)])

        @pl.loop(0, sb, step=2)
        def _(j):
            pltpu.async_copy(ones_v, acc.at[didx.at[j]], t0, add=True)
            pltpu.async_copy(ones_v, acc.at[didx.at[j + 1]], t1, add=True)
            pltpu.make_async_copy(ones_v, acc.at[didx.at[j]], t0).wait()
            pltpu.make_async_copy(ones_v, acc.at[didx.at[j + 1]], t1).wait()

    plsc.subcore_barrier()
    pltpu.sync_copy(acc.at[pl.ds(sid * rps, rps)],
                    out.at[pl.ds(cid * n_dst_pad + sid * rps, rps)])


@functools.lru_cache(maxsize=None)
def _make_counts(np_map, e_pad_map):
    np_map = dict(np_map)
    e_pad_map = dict(e_pad_map)
    mesh = plsc.VectorSubcoreMesh(core_axis_name="c", subcore_axis_name="s")
    big_rps = np_map["vals"] // NSUB
    small_rps = np_map["obj"] // NSUB

    def body(d0_, d1_, d2_, d3_, d4_, d5_, ones_h, zeros_big, zeros_small,
             o0, o1, o2, o3, o4, o5,
             ones_v, didx, acc_big, acc_small, t0, t1):
        cid = lax.axis_index("c")
        sid = lax.axis_index("s")
        wid = sid * NCORE + cid
        pltpu.sync_copy(ones_h, ones_v)
        idx = [d0_, d1_, d2_, d3_, d4_, d5_]
        outs = [o0, o1, o2, o3, o4, o5]
        for k, (name, st, dt) in enumerate(_PHASES):
            big = np_map[dt] > 128
            _cnt_phase(idx[k], zeros_big if big else zeros_small, outs[k],
                       acc_big if big else acc_small, ones_v, wid, cid, sid,
                       didx, t0, t1, np_map[dt], e_pad_map[name])

    kern = pl.kernel(
        body,
        out_type=tuple(
            jax.ShapeDtypeStruct((NCORE * np_map[dt], CWSC), jnp.float32)
            for _, _, dt in _PHASES),
        mesh=mesh,
        scratch_types=[
            pltpu.VMEM((CHUNK, CWSC), jnp.float32),
            pltpu.VMEM((40, CHUNK), jnp.int32),
            pltpu.VMEM_SHARED((np_map["vals"], CWSC), jnp.float32),
            pltpu.VMEM_SHARED((np_map["obj"], CWSC), jnp.float32),
            pltpu.SemaphoreType.DMA,
            pltpu.SemaphoreType.DMA,
        ],
    )

    def call(prep):
        args = []
        for name, st, dt in _PHASES:
            src_p, dst_p, e_pad = prep[name]
            args.append(dst_p.reshape(e_pad // CHUNK, CHUNK))
        args.append(jnp.ones((CHUNK, CWSC), jnp.float32))
        args.append(jnp.zeros((big_rps, CWSC), jnp.float32))
        args.append(jnp.zeros((small_rps, CWSC), jnp.float32))
        outs = kern(*args)
        return {name: outs[k].reshape(NCORE, np_map[dt], CWSC)[:, :, :CW]
                for k, (name, st, dt) in enumerate(_PHASES)}

    return call


def _dot(a, b):
    return jnp.dot(a, b, preferred_element_type=jnp.float32)


def _tc_enc(x, p):
    (w1, b1), (w2, b2) = p
    b1 = b1.reshape(1, -1)
    b2 = b2.reshape(1, -1)
    npad = x.shape[0]
    br = min(npad, 2048)

    def body(x_ref, w1_ref, b1_ref, w2_ref, b2_ref, o_ref):
        t = jnp.maximum(_dot(x_ref[...], w1_ref[...]) + b1_ref[...], 0.0)
        o_ref[...] = _dot(t, w2_ref[...]) + b2_ref[...]

    full = lambda a: pl.BlockSpec(a.shape, lambda i: (0,) * a.ndim)
    return pl.pallas_call(
        body,
        grid=(npad // br,),
        in_specs=[
            pl.BlockSpec((br, x.shape[1]), lambda i: (i, 0)),
            full(w1), full(b1), full(w2), full(b2),
        ],
        out_specs=pl.BlockSpec((br, D), lambda i: (i, 0)),
        out_shape=jax.ShapeDtypeStruct((npad, D), jnp.float32),
    )(x, w1, b1, w2, b2)


def _tc_update(s1, c1, p1, s2, c2, p2, h):
    (w11, b11), (w12, b12) = p1
    (w21, b21), (w22, b22) = p2
    b11, b12, b21, b22 = (b.reshape(1, -1) for b in (b11, b12, b21, b22))
    npad = h.shape[0]
    br = min(npad, 2048)

    def body(s1_ref, c1_ref, w11_r, b11_r, w12_r, b12_r,
             s2_ref, c2_ref, w21_r, b21_r, w22_r, b22_r, h_ref, o_ref):
        def half(s_ref, c_ref, wa, ba, wb, bb):
            s = s_ref[0] + s_ref[1]
            c = jnp.maximum(c_ref[0, :, 0:1] + c_ref[1, :, 0:1], 1.0)
            t = jnp.maximum(_dot(s / c, wa[...]) + ba[...], 0.0)
            return _dot(t, wb[...]) + bb[...]

        z = jnp.concatenate(
            [half(s1_ref, c1_ref, w11_r, b11_r, w12_r, b12_r),
             half(s2_ref, c2_ref, w21_r, b21_r, w22_r, b22_r)], axis=1)
        o_ref[...] = (jnp.maximum(z, 0.0) + h_ref[...]) * 0.5

    full = lambda a: pl.BlockSpec(a.shape, lambda i: (0,) * a.ndim)
    sspec = pl.BlockSpec((2, br, D), lambda i: (0, i, 0))
    cspec = pl.BlockSpec((2, br, CW), lambda i: (0, i, 0))
    hspec = pl.BlockSpec((br, D), lambda i: (i, 0))
    return pl.pallas_call(
        body,
        grid=(npad // br,),
        in_specs=[sspec, cspec, full(w11), full(b11), full(w12), full(b12),
                  sspec, cspec, full(w21), full(b21), full(w22), full(b22),
                  hspec],
        out_specs=hspec,
        out_shape=jax.ShapeDtypeStruct((npad, D), jnp.float32),
    )(s1, c1, w11, b11, w12, b12, s2, c2, w21, b21, w22, b22, h)


def _tc_pred(h1, h2, p, do_relu):
    (w1, b1), (w2, b2) = p
    b1 = b1.reshape(1, -1)
    w2 = jnp.pad(w2, ((0, 0), (0, 8 - w2.shape[1])))
    b2 = jnp.pad(b2.reshape(1, -1), ((0, 0), (0, 8 - b2.shape[0])))
    npad = h1.shape[0]
    br = min(npad, 2048)

    def body(h1_ref, h2_ref, w1_r, b1_r, w2_r, b2_r, o_ref):
        def one(h_ref):
            t = jnp.maximum(_dot(h_ref[...], w1_r[...]) + b1_r[...], 0.0)
            return _dot(t, w2_r[...]) + b2_r[...]

        z = jnp.concatenate([one(h1_ref), one(h2_ref)], axis=1)
        if do_relu:
            z = jnp.maximum(z, 0.0)
        o_ref[...] = z

    full = lambda a: pl.BlockSpec(a.shape, lambda i: (0,) * a.ndim)
    hspec = pl.BlockSpec((br, D), lambda i: (i, 0))
    return pl.pallas_call(
        body,
        grid=(npad // br,),
        in_specs=[hspec, hspec, full(w1), full(b1), full(w2), full(b2)],
        out_specs=pl.BlockSpec((br, 16), lambda i: (i, 0)),
        out_shape=jax.ShapeDtypeStruct((npad, 16), jnp.float32),
    )(h1, h2, w1, b1, w2, b2)


_N = {"cons": 10000, "vals": 10000, "obj": 100}
_NP = {"cons": 10240, "vals": 10240, "obj": 128}
_REL = {
    "cons_to_vals": ("cons", "vals"),
    "vals_to_cons": ("vals", "cons"),
    "vals_to_obj": ("vals", "obj"),
    "obj_to_vals": ("obj", "vals"),
    "cons_to_obj": ("cons", "obj"),
    "obj_to_cons": ("obj", "cons"),
}


def kernel(x_cons, x_vals, x_obj, params, e_cons_to_vals, e_vals_to_cons,
           e_vals_to_obj, e_obj_to_vals, e_cons_to_obj, e_obj_to_cons):
    edges = {
        "cons_to_vals": e_cons_to_vals, "vals_to_cons": e_vals_to_cons,
        "vals_to_obj": e_vals_to_obj, "obj_to_vals": e_obj_to_vals,
        "cons_to_obj": e_cons_to_obj, "obj_to_cons": e_obj_to_cons,
    }
    x = {"cons": x_cons, "vals": x_vals, "obj": x_obj}

    prep = {}
    for name, (s, d) in _REL.items():
        src, dst = edges[name][0], edges[name][1]
        e = src.shape[0]
        e_pad = _round_up(e, NW * CHUNK * DEPTH)
        pad = e_pad - e
        src_p = jnp.concatenate(
            [src, jnp.arange(pad, dtype=jnp.int32) % _N[s]])
        dst_p = jnp.concatenate(
            [dst, _N[d] + (jnp.arange(pad, dtype=jnp.int32) % (_NP[d] - _N[d]))])
        prep[name] = (src_p, dst_p, e_pad)

    np_key = tuple(sorted(_NP.items()))
    ep_key = tuple(sorted((n, prep[n][2]) for n in prep))
    counts = _make_counts(np_key, ep_key)(prep)

    h = {}
    for t in ("cons", "vals", "obj"):
        xp = jnp.pad(x[t], ((0, _NP[t] - _N[t]), (0, 0)))
        h[t] = _tc_enc(xp, params["enc"][t])

    vals_list, cons_list = [], []
    groups = {"vals": _PH_V, "obj": _PH_O, "cons": _PH_C}
    rotation = ["vals", "obj", "cons"]
    round_idx = 0
    for _k in range(2):
        for j in range(2):
            pj = params["conv"][j]
            order = rotation[-round_idx % 3:] + rotation[:-round_idx % 3]
            sums = {}
            for g in order:
                sums.update(_make_round(np_key, ep_key, groups[g])(h, prep))
            h = {
                "vals": _tc_update(sums["cons_to_vals"], counts["cons_to_vals"],
                                   pj["cons_to_vals"], sums["obj_to_vals"],
                                   counts["obj_to_vals"], pj["obj_to_vals"],
                                   h["vals"]),
                "cons": _tc_update(sums["vals_to_cons"], counts["vals_to_cons"],
                                   pj["vals_to_cons"], sums["obj_to_cons"],
                                   counts["obj_to_cons"], pj["obj_to_cons"],
                                   h["cons"]),
                "obj": _tc_update(sums["vals_to_obj"], counts["vals_to_obj"],
                                  pj["vals_to_obj"], sums["cons_to_obj"],
                                  counts["cons_to_obj"], pj["cons_to_obj"],
                                  h["obj"]),
            }
            round_idx += 1
        vals_list.append(h["vals"])
        cons_list.append(h["cons"])

    pv = _tc_pred(vals_list[0], vals_list[1], params["pred_vals"], True)
    pc = _tc_pred(cons_list[0], cons_list[1], params["pred_cons"], False)
    vals = jnp.stack([pv[:_N["vals"], 0], pv[:_N["vals"], 8]], axis=1)
    cons = jnp.stack([pc[:_N["cons"], 0], pc[:_N["cons"], 8]], axis=1)
    return (vals, cons)


# speedup vs baseline: 1.0213x; 1.0213x over previous
import functools

import jax
import jax.numpy as jnp
from jax import lax
from jax.experimental import pallas as pl
from jax.experimental.pallas import tpu as pltpu
from jax.experimental.pallas import tpu_sc as plsc

D = 128
HID = 64
NCORE = 2
NSUB = 16
NW = NCORE * NSUB
CHUNK = 128
CW = 16
CWSC = 128


def _round_up(x, m):
    return (x + m - 1) // m * m


DEPTH = 4


def _seg_phase(table, src2, dst2, zeros_h, out, acc, wid, cid, sid,
               sidx, didx, rows0, rows1, g0, g1, t0, t1,
               n_dst_pad, e_pad):
    n_chunks = e_pad // NW // CHUNK
    sb = 40 if n_chunks % 40 == 0 else n_chunks
    n_sb = n_chunks // sb
    rps = n_dst_pad // NSUB

    pltpu.sync_copy(zeros_h, acc.at[pl.ds(sid * rps, rps)])
    plsc.subcore_barrier()

    @pl.loop(0, n_sb)
    def _(b):
        c0 = wid * n_chunks + b * sb
        pltpu.sync_copy(src2.at[pl.ds(c0, sb)], sidx.at[pl.ds(0, sb)])
        pltpu.sync_copy(dst2.at[pl.ds(c0, sb)], didx.at[pl.ds(0, sb)])

        pltpu.async_copy(table.at[sidx.at[0]], rows0, g0)
        pltpu.async_copy(table.at[sidx.at[1]], rows1, g1)

        @pl.loop(0, sb - 2, step=2)
        def _(j):
            pltpu.make_async_copy(table.at[sidx.at[j]], rows0, g0).wait()
            pltpu.async_copy(rows0, acc.at[didx.at[j]], t0, add=True)
            pltpu.make_async_copy(table.at[sidx.at[j + 1]], rows1, g1).wait()
            pltpu.async_copy(rows1, acc.at[didx.at[j + 1]], t1, add=True)
            pltpu.make_async_copy(rows0, acc.at[didx.at[j]], t0).wait()
            pltpu.async_copy(table.at[sidx.at[j + 2]], rows0, g0)
            pltpu.make_async_copy(rows1, acc.at[didx.at[j + 1]], t1).wait()
            pltpu.async_copy(table.at[sidx.at[j + 3]], rows1, g1)

        pltpu.make_async_copy(table.at[sidx.at[sb - 2]], rows0, g0).wait()
        pltpu.sync_copy(rows0, acc.at[didx.at[sb - 2]], add=True)
        pltpu.make_async_copy(table.at[sidx.at[sb - 1]], rows1, g1).wait()
        pltpu.sync_copy(rows1, acc.at[didx.at[sb - 1]], add=True)

    plsc.subcore_barrier()
    pltpu.sync_copy(acc.at[pl.ds(sid * rps, rps)],
                    out.at[pl.ds(cid * n_dst_pad + sid * rps, rps)])


_PHASES_A = (
    ("cons_to_vals", "cons", "vals"),
    ("obj_to_vals", "obj", "vals"),
    ("vals_to_obj", "vals", "obj"),
    ("cons_to_obj", "cons", "obj"),
)
_PHASES_B = (
    ("vals_to_cons", "vals", "cons"),
    ("obj_to_cons", "obj", "cons"),
)
_PHASES = _PHASES_A + _PHASES_B


@functools.lru_cache(maxsize=None)
def _make_round(np_map, e_pad_map, phases):
    np_map = dict(np_map)
    e_pad_map = dict(e_pad_map)
    mesh = plsc.VectorSubcoreMesh(core_axis_name="c", subcore_axis_name="s")
    big_rps = np_map["vals"] // NSUB
    small_rps = np_map["obj"] // NSUB
    n_ph = len(phases)
    has_small = any(np_map[dt] <= 128 for _, _, dt in phases)

    def body(*refs):
        t_cons, t_vals, t_obj = refs[0:3]
        idx = [(refs[3 + 2 * k], refs[4 + 2 * k]) for k in range(n_ph)]
        zeros_big, zeros_small = refs[3 + 2 * n_ph], refs[4 + 2 * n_ph]
        outs = refs[5 + 2 * n_ph:5 + 3 * n_ph]
        (sidx, didx, rows0, rows1, acc_big, acc_small,
         g0, g1, t0, t1) = refs[5 + 3 * n_ph:]
        cid = lax.axis_index("c")
        sid = lax.axis_index("s")
        wid = sid * NCORE + cid
        tables = {"cons": t_cons, "vals": t_vals, "obj": t_obj}
        for k, (name, st, dt) in enumerate(phases):
            big = np_map[dt] > 128
            _seg_phase(tables[st], idx[k][0], idx[k][1],
                       zeros_big if big else zeros_small, outs[k],
                       acc_big if big else acc_small, wid, cid, sid,
                       sidx, didx, rows0, rows1, g0, g1, t0, t1,
                       np_map[dt], e_pad_map[name])

    kern = pl.kernel(
        body,
        out_type=tuple(
            jax.ShapeDtypeStruct((NCORE * np_map[dt], D), jnp.float32)
            for _, _, dt in phases),
        mesh=mesh,
        scratch_types=[
            pltpu.VMEM((40, CHUNK), jnp.int32),
            pltpu.VMEM((40, CHUNK), jnp.int32),
            pltpu.VMEM((CHUNK, D), jnp.float32),
            pltpu.VMEM((CHUNK, D), jnp.float32),
            pltpu.VMEM_SHARED((np_map["vals"], D), jnp.float32),
            pltpu.VMEM_SHARED((np_map["obj"] if has_small else 8, D),
                              jnp.float32),
            pltpu.SemaphoreType.DMA,
            pltpu.SemaphoreType.DMA,
            pltpu.SemaphoreType.DMA,
            pltpu.SemaphoreType.DMA,
        ],
    )

    def call(h, prep):
        args = [h["cons"], h["vals"], h["obj"]]
        for name, st, dt in phases:
            src_p, dst_p, e_pad = prep[name]
            args.append(src_p.reshape(e_pad // CHUNK, CHUNK))
            args.append(dst_p.reshape(e_pad // CHUNK, CHUNK))
        args.append(jnp.zeros((big_rps, D), jnp.float32))
        args.append(jnp.zeros((small_rps, D), jnp.float32))
        outs = kern(*args)
        return {name: outs[k].reshape(NCORE, np_map[dt], D)
                for k, (name, st, dt) in enumerate(phases)}

    return call


def _cnt_phase(dst2, zeros_h, out, acc, ones_v, wid, cid, sid, didx,
               t0, t1, n_dst_pad, e_pad):
    n_chunks = e_pad // NW // CHUNK
    sb = 40 if n_chunks % 40 == 0 else n_chunks
    n_sb = n_chunks // sb
    rps = n_dst_pad // NSUB

    pltpu.sync_copy(zeros_h, acc.at[pl.ds(sid * rps, rps)])
    plsc.subcore_barrier()

    @pl.loop(0, n_sb)
    def _(b):
        c0 = wid * n_chunks + b * sb
        pltpu.sync_copy(dst2.at[pl.ds(c0, sb)], didx.at[pl.ds(0, sb)])

        @pl.loop(0, sb, step=2)
        def _(j):
            pltpu.async_copy(ones_v, acc.at[didx.at[j]], t0, add=True)
            pltpu.async_copy(ones_v, acc.at[didx.at[j + 1]], t1, add=True)
            pltpu.make_async_copy(ones_v, acc.at[didx.at[j]], t0).wait()
            pltpu.make_async_copy(ones_v, acc.at[didx.at[j + 1]], t1).wait()

    plsc.subcore_barrier()
    pltpu.sync_copy(acc.at[pl.ds(sid * rps, rps)],
                    out.at[pl.ds(cid * n_dst_pad + sid * rps, rps)])


@functools.lru_cache(maxsize=None)
def _make_counts(np_map, e_pad_map):
    np_map = dict(np_map)
    e_pad_map = dict(e_pad_map)
    mesh = plsc.VectorSubcoreMesh(core_axis_name="c", subcore_axis_name="s")
    big_rps = np_map["vals"] // NSUB
    small_rps = np_map["obj"] // NSUB

    def body(d0_, d1_, d2_, d3_, d4_, d5_, ones_h, zeros_big, zeros_small,
             o0, o1, o2, o3, o4, o5,
             ones_v, didx, acc_big, acc_small, t0, t1):
        cid = lax.axis_index("c")
        sid = lax.axis_index("s")
        wid = sid * NCORE + cid
        pltpu.sync_copy(ones_h, ones_v)
        idx = [d0_, d1_, d2_, d3_, d4_, d5_]
        outs = [o0, o1, o2, o3, o4, o5]
        for k, (name, st, dt) in enumerate(_PHASES):
            big = np_map[dt] > 128
            _cnt_phase(idx[k], zeros_big if big else zeros_small, outs[k],
                       acc_big if big else acc_small, ones_v, wid, cid, sid,
                       didx, t0, t1, np_map[dt], e_pad_map[name])

    kern = pl.kernel(
        body,
        out_type=tuple(
            jax.ShapeDtypeStruct((NCORE * np_map[dt], CWSC), jnp.float32)
            for _, _, dt in _PHASES),
        mesh=mesh,
        scratch_types=[
            pltpu.VMEM((CHUNK, CWSC), jnp.float32),
            pltpu.VMEM((40, CHUNK), jnp.int32),
            pltpu.VMEM_SHARED((np_map["vals"], CWSC), jnp.float32),
            pltpu.VMEM_SHARED((np_map["obj"], CWSC), jnp.float32),
            pltpu.SemaphoreType.DMA,
            pltpu.SemaphoreType.DMA,
        ],
    )

    def call(prep):
        args = []
        for name, st, dt in _PHASES:
            src_p, dst_p, e_pad = prep[name]
            args.append(dst_p.reshape(e_pad // CHUNK, CHUNK))
        args.append(jnp.ones((CHUNK, CWSC), jnp.float32))
        args.append(jnp.zeros((big_rps, CWSC), jnp.float32))
        args.append(jnp.zeros((small_rps, CWSC), jnp.float32))
        outs = kern(*args)
        return {name: outs[k].reshape(NCORE, np_map[dt], CWSC)[:, :, :CW]
                for k, (name, st, dt) in enumerate(_PHASES)}

    return call


def _dot(a, b):
    return jnp.dot(a, b, preferred_element_type=jnp.float32)


def _tc_enc(x, p):
    (w1, b1), (w2, b2) = p
    b1 = b1.reshape(1, -1)
    b2 = b2.reshape(1, -1)
    npad = x.shape[0]
    br = min(npad, 2048)

    def body(x_ref, w1_ref, b1_ref, w2_ref, b2_ref, o_ref):
        t = jnp.maximum(_dot(x_ref[...], w1_ref[...]) + b1_ref[...], 0.0)
        o_ref[...] = _dot(t, w2_ref[...]) + b2_ref[...]

    full = lambda a: pl.BlockSpec(a.shape, lambda i: (0,) * a.ndim)
    return pl.pallas_call(
        body,
        grid=(npad // br,),
        in_specs=[
            pl.BlockSpec((br, x.shape[1]), lambda i: (i, 0)),
            full(w1), full(b1), full(w2), full(b2),
        ],
        out_specs=pl.BlockSpec((br, D), lambda i: (i, 0)),
        out_shape=jax.ShapeDtypeStruct((npad, D), jnp.float32),
    )(x, w1, b1, w2, b2)


def _tc_update(s1, c1, p1, s2, c2, p2, h):
    (w11, b11), (w12, b12) = p1
    (w21, b21), (w22, b22) = p2
    b11, b12, b21, b22 = (b.reshape(1, -1) for b in (b11, b12, b21, b22))
    npad = h.shape[0]
    br = min(npad, 2048)

    def body(s1_ref, c1_ref, w11_r, b11_r, w12_r, b12_r,
             s2_ref, c2_ref, w21_r, b21_r, w22_r, b22_r, h_ref, o_ref):
        def half(s_ref, c_ref, wa, ba, wb, bb):
            s = s_ref[0] + s_ref[1]
            c = jnp.maximum(c_ref[0, :, 0:1] + c_ref[1, :, 0:1], 1.0)
            t = jnp.maximum(_dot(s / c, wa[...]) + ba[...], 0.0)
            return _dot(t, wb[...]) + bb[...]

        z = jnp.concatenate(
            [half(s1_ref, c1_ref, w11_r, b11_r, w12_r, b12_r),
             half(s2_ref, c2_ref, w21_r, b21_r, w22_r, b22_r)], axis=1)
        o_ref[...] = (jnp.maximum(z, 0.0) + h_ref[...]) * 0.5

    full = lambda a: pl.BlockSpec(a.shape, lambda i: (0,) * a.ndim)
    sspec = pl.BlockSpec((2, br, D), lambda i: (0, i, 0))
    cspec = pl.BlockSpec((2, br, CW), lambda i: (0, i, 0))
    hspec = pl.BlockSpec((br, D), lambda i: (i, 0))
    return pl.pallas_call(
        body,
        grid=(npad // br,),
        in_specs=[sspec, cspec, full(w11), full(b11), full(w12), full(b12),
                  sspec, cspec, full(w21), full(b21), full(w22), full(b22),
                  hspec],
        out_specs=hspec,
        out_shape=jax.ShapeDtypeStruct((npad, D), jnp.float32),
    )(s1, c1, w11, b11, w12, b12, s2, c2, w21, b21, w22, b22, h)


def _tc_pred(h1, h2, p, do_relu):
    (w1, b1), (w2, b2) = p
    b1 = b1.reshape(1, -1)
    w2 = jnp.pad(w2, ((0, 0), (0, 8 - w2.shape[1])))
    b2 = jnp.pad(b2.reshape(1, -1), ((0, 0), (0, 8 - b2.shape[0])))
    npad = h1.shape[0]
    br = min(npad, 2048)

    def body(h1_ref, h2_ref, w1_r, b1_r, w2_r, b2_r, o_ref):
        def one(h_ref):
            t = jnp.maximum(_dot(h_ref[...], w1_r[...]) + b1_r[...], 0.0)
            return _dot(t, w2_r[...]) + b2_r[...]

        z = jnp.concatenate([one(h1_ref), one(h2_ref)], axis=1)
        if do_relu:
            z = jnp.maximum(z, 0.0)
        o_ref[...] = z

    full = lambda a: pl.BlockSpec(a.shape, lambda i: (0,) * a.ndim)
    hspec = pl.BlockSpec((br, D), lambda i: (i, 0))
    return pl.pallas_call(
        body,
        grid=(npad // br,),
        in_specs=[hspec, hspec, full(w1), full(b1), full(w2), full(b2)],
        out_specs=pl.BlockSpec((br, 16), lambda i: (i, 0)),
        out_shape=jax.ShapeDtypeStruct((npad, 16), jnp.float32),
    )(h1, h2, w1, b1, w2, b2)


_N = {"cons": 10000, "vals": 10000, "obj": 100}
_NP = {"cons": 10240, "vals": 10240, "obj": 128}
_REL = {
    "cons_to_vals": ("cons", "vals"),
    "vals_to_cons": ("vals", "cons"),
    "vals_to_obj": ("vals", "obj"),
    "obj_to_vals": ("obj", "vals"),
    "cons_to_obj": ("cons", "obj"),
    "obj_to_cons": ("obj", "cons"),
}


def kernel(x_cons, x_vals, x_obj, params, e_cons_to_vals, e_vals_to_cons,
           e_vals_to_obj, e_obj_to_vals, e_cons_to_obj, e_obj_to_cons):
    edges = {
        "cons_to_vals": e_cons_to_vals, "vals_to_cons": e_vals_to_cons,
        "vals_to_obj": e_vals_to_obj, "obj_to_vals": e_obj_to_vals,
        "cons_to_obj": e_cons_to_obj, "obj_to_cons": e_obj_to_cons,
    }
    x = {"cons": x_cons, "vals": x_vals, "obj": x_obj}

    prep = {}
    for name, (s, d) in _REL.items():
        src, dst = edges[name][0], edges[name][1]
        e = src.shape[0]
        e_pad = _round_up(e, NW * CHUNK * DEPTH)
        pad = e_pad - e
        src_p = jnp.concatenate(
            [src, jnp.arange(pad, dtype=jnp.int32) % _N[s]])
        dst_p = jnp.concatenate(
            [dst, _N[d] + (jnp.arange(pad, dtype=jnp.int32) % (_NP[d] - _N[d]))])
        prep[name] = (src_p, dst_p, e_pad)

    np_key = tuple(sorted(_NP.items()))
    ep_key = tuple(sorted((n, prep[n][2]) for n in prep))
    counts = _make_counts(np_key, ep_key)(prep)

    h = {}
    for t in ("cons", "vals", "obj"):
        xp = jnp.pad(x[t], ((0, _NP[t] - _N[t]), (0, 0)))
        h[t] = _tc_enc(xp, params["enc"][t])

    vals_list, cons_list = [], []
    for _k in range(2):
        for j in range(2):
            pj = params["conv"][j]
            sums_a = _make_round(np_key, ep_key, _PHASES_A)(h, prep)
            new_vals = _tc_update(sums_a["cons_to_vals"],
                                  counts["cons_to_vals"], pj["cons_to_vals"],
                                  sums_a["obj_to_vals"], counts["obj_to_vals"],
                                  pj["obj_to_vals"], h["vals"])
            new_obj = _tc_update(sums_a["vals_to_obj"], counts["vals_to_obj"],
                                 pj["vals_to_obj"], sums_a["cons_to_obj"],
                                 counts["cons_to_obj"], pj["cons_to_obj"],
                                 h["obj"])
            sums_b = _make_round(np_key, ep_key, _PHASES_B)(h, prep)
            h = {
                "vals": new_vals,
                "obj": new_obj,
                "cons": _tc_update(sums_b["vals_to_cons"],
                                   counts["vals_to_cons"], pj["vals_to_cons"],
                                   sums_b["obj_to_cons"], counts["obj_to_cons"],
                                   pj["obj_to_cons"], h["cons"]),
            }
        vals_list.append(h["vals"])
        cons_list.append(h["cons"])

    pv = _tc_pred(vals_list[0], vals_list[1], params["pred_vals"], True)
    pc = _tc_pred(cons_list[0], cons_list[1], params["pred_cons"], False)
    vals = jnp.stack([pv[:_N["vals"], 0], pv[:_N["vals"], 8]], axis=1)
    cons = jnp.stack([pc[:_N["cons"], 0], pc[:_N["cons"], 8]], axis=1)
    return (vals, cons)
